# Initial kernel scaffold; baseline (speedup 1.0000x reference)
import jax
import jax.numpy as jnp
from jax.experimental import pallas as pl


def kernel(x, edge_index, W1, b1):
    raise NotImplementedError("write your pallas kernel here")


# speedup vs baseline: 26.7951x; 26.7951x over previous
import functools

import jax
import jax.numpy as jnp
from jax import lax
from jax.experimental import pallas as pl
from jax.experimental.pallas import tpu as pltpu
from jax.experimental.pallas import tpu_sc as plsc

N_NODES = 10000
N_EDGES = 320000
D_FEAT = 128
HIDDEN = 16

NC = 2
NS = 16
LANES = 16

CHUNK = 128
N_CHUNKS = N_EDGES // CHUNK
CHUNKS_PER_TILE = N_CHUNKS // (NC * NS)
EXTRA_CHUNKS = N_CHUNKS - CHUNKS_PER_TILE * NC * NS
ROWS_PER_SUB = 624
ROWS_TAIL = N_NODES - NS * ROWS_PER_SUB

_sc_mesh = plsc.VectorSubcoreMesh(
    core_axis_name="c", subcore_axis_name="s", num_cores=NC, num_subcores=NS
)

_sc_params = pltpu.CompilerParams(use_tc_tiling_on_sc=False)


@functools.partial(
    pl.kernel,
    out_type=jax.ShapeDtypeStruct((NC, N_NODES, HIDDEN), jnp.float32),
    mesh=_sc_mesh,
    compiler_params=_sc_params,
    scratch_types=[
        pltpu.VMEM((CHUNK,), jnp.int32),
        pltpu.VMEM((CHUNK, HIDDEN), jnp.float32),
        pltpu.VMEM((ROWS_PER_SUB, HIDDEN), jnp.float32),
        pltpu.VMEM_SHARED((N_NODES, HIDDEN), jnp.float32),
    ],
)
def _hist(dst_hbm, out_hbm, idx_v, ones_v, zrows_v, acc_sh):
    c = lax.axis_index("c")
    s = lax.axis_index("s")
    wid = s * NC + c

    @pl.loop(0, CHUNK)
    def _(i):
        ones_v[i, :] = jnp.ones((LANES,), jnp.float32)

    @pl.loop(0, ROWS_PER_SUB)
    def _(i):
        zrows_v[i, :] = jnp.zeros((LANES,), jnp.float32)

    pltpu.sync_copy(zrows_v, acc_sh.at[pl.ds(s * ROWS_PER_SUB, ROWS_PER_SUB)])

    @pl.when(s == NS - 1)
    def _():
        pltpu.sync_copy(zrows_v.at[pl.ds(0, ROWS_TAIL)],
                        acc_sh.at[pl.ds(NS * ROWS_PER_SUB, ROWS_TAIL)])

    plsc.subcore_barrier()

    def process(block):
        off = block * CHUNK
        pltpu.sync_copy(dst_hbm.at[pl.ds(off, CHUNK)], idx_v)
        pltpu.sync_copy(ones_v, acc_sh.at[idx_v], add=True)

    @pl.loop(0, CHUNKS_PER_TILE)
    def _(k):
        process(wid * CHUNKS_PER_TILE + k)

    @pl.when(wid < EXTRA_CHUNKS)
    def _():
        process(NC * NS * CHUNKS_PER_TILE + wid)

    plsc.subcore_barrier()
    pltpu.sync_copy(
        acc_sh.at[pl.ds(s * ROWS_PER_SUB, ROWS_PER_SUB)],
        out_hbm.at[c, pl.ds(s * ROWS_PER_SUB, ROWS_PER_SUB)],
    )

    @pl.when(s == NS - 1)
    def _():
        pltpu.sync_copy(
            acc_sh.at[pl.ds(NS * ROWS_PER_SUB, ROWS_TAIL)],
            out_hbm.at[c, pl.ds(NS * ROWS_PER_SUB, ROWS_TAIL)],
        )


@functools.partial(
    pl.kernel,
    out_type=jax.ShapeDtypeStruct((NC, N_NODES, HIDDEN), jnp.float32),
    mesh=_sc_mesh,
    compiler_params=_sc_params,
    scratch_types=[
        pltpu.VMEM((CHUNK,), jnp.int32),
        pltpu.VMEM((CHUNK,), jnp.int32),
        pltpu.VMEM((CHUNK, HIDDEN), jnp.float32),
        pltpu.VMEM((ROWS_PER_SUB, HIDDEN), jnp.float32),
        pltpu.VMEM_SHARED((N_NODES, HIDDEN), jnp.float32),
        pltpu.SemaphoreType.DMA,
    ],
)
def _segsum(src_hbm, dst_hbm, y_hbm, out_hbm,
            idx_s_v, idx_d_v, rows_v, zrows_v, acc_sh, sem):
    c = lax.axis_index("c")
    s = lax.axis_index("s")
    wid = s * NC + c

    @pl.loop(0, ROWS_PER_SUB)
    def _(i):
        zrows_v[i, :] = jnp.zeros((LANES,), jnp.float32)

    rows_slc = pl.ds(s * ROWS_PER_SUB, ROWS_PER_SUB)
    tail_slc = pl.ds(NS * ROWS_PER_SUB, ROWS_TAIL)
    pltpu.sync_copy(zrows_v, acc_sh.at[rows_slc])

    @pl.when(s == NS - 1)
    def _():
        pltpu.sync_copy(zrows_v.at[pl.ds(0, ROWS_TAIL)], acc_sh.at[tail_slc])

    plsc.subcore_barrier()

    def process(block):
        off = block * CHUNK
        pltpu.sync_copy(src_hbm.at[pl.ds(off, CHUNK)], idx_s_v)
        pltpu.sync_copy(dst_hbm.at[pl.ds(off, CHUNK)], idx_d_v)
        pltpu.async_copy(y_hbm.at[idx_s_v], rows_v, sem).wait()
        pltpu.sync_copy(rows_v, acc_sh.at[idx_d_v], add=True)

    @pl.loop(0, CHUNKS_PER_TILE)
    def _(k):
        process(wid * CHUNKS_PER_TILE + k)

    @pl.when(wid < EXTRA_CHUNKS)
    def _():
        process(NC * NS * CHUNKS_PER_TILE + wid)

    plsc.subcore_barrier()
    pltpu.sync_copy(acc_sh.at[rows_slc], out_hbm.at[c, rows_slc])

    @pl.when(s == NS - 1)
    def _():
        pltpu.sync_copy(acc_sh.at[tail_slc], out_hbm.at[c, tail_slc])


_BLK = 1000


def _dense_body(x_ref, w_ref, degp_ref, y_ref):
    deg = degp_ref[0] + degp_ref[1] + 1.0
    dis = lax.rsqrt(deg)
    xw = jnp.dot(x_ref[...], w_ref[...], preferred_element_type=jnp.float32)
    y_ref[...] = xw * dis


def _dense(x, w, degp):
    return pl.pallas_call(
        _dense_body,
        grid=(N_NODES // _BLK,),
        in_specs=[
            pl.BlockSpec((_BLK, D_FEAT), lambda i: (i, 0)),
            pl.BlockSpec((D_FEAT, HIDDEN), lambda i: (0, 0)),
            pl.BlockSpec((NC, _BLK, HIDDEN), lambda i: (0, i, 0)),
        ],
        out_specs=pl.BlockSpec((_BLK, HIDDEN), lambda i: (i, 0)),
        out_shape=jax.ShapeDtypeStruct((N_NODES, HIDDEN), jnp.float32),
    )(x, w, degp)


def _final_body(sp_ref, y_ref, degp_ref, b_ref, o_ref):
    deg = degp_ref[0] + degp_ref[1] + 1.0
    dis = lax.rsqrt(deg)
    h = dis * (sp_ref[0] + sp_ref[1] + y_ref[...]) + b_ref[...]
    h = jnp.maximum(h, 0.0)
    m = jnp.max(h, axis=1, keepdims=True)
    lse = jnp.log(jnp.sum(jnp.exp(h - m), axis=1, keepdims=True))
    o_ref[...] = h - m - lse


def _final(sp, y, degp, b):
    return pl.pallas_call(
        _final_body,
        grid=(N_NODES // _BLK,),
        in_specs=[
            pl.BlockSpec((NC, _BLK, HIDDEN), lambda i: (0, i, 0)),
            pl.BlockSpec((_BLK, HIDDEN), lambda i: (i, 0)),
            pl.BlockSpec((NC, _BLK, HIDDEN), lambda i: (0, i, 0)),
            pl.BlockSpec((1, HIDDEN), lambda i: (0, 0)),
        ],
        out_specs=pl.BlockSpec((_BLK, HIDDEN), lambda i: (i, 0)),
        out_shape=jax.ShapeDtypeStruct((N_NODES, HIDDEN), jnp.float32),
    )(sp, y, degp, b)


@jax.jit
def kernel(x, edge_index, W1, b1):
    src = edge_index[0].astype(jnp.int32)
    dst = edge_index[1].astype(jnp.int32)
    degp = _hist(dst)
    y = _dense(x, W1, degp)
    sp = _segsum(src, dst, y)
    return _final(sp, y, degp, b1.reshape(1, HIDDEN))


# speedup vs baseline: 53.6195x; 2.0011x over previous
import functools

import jax
import jax.numpy as jnp
from jax import lax
from jax.experimental import pallas as pl
from jax.experimental.pallas import tpu as pltpu
from jax.experimental.pallas import tpu_sc as plsc

N_NODES = 10000
N_EDGES = 320000
D_FEAT = 128
HIDDEN = 16

NC = 2
NS = 16
LANES = 16

CHUNK = 128
N_CHUNKS = N_EDGES // CHUNK
NB = N_CHUNKS // (NC * NS)
NPAIR = NB // 2
EXTRA_CHUNKS = N_CHUNKS - NB * NC * NS
ROWS_PER_SUB = 624
ROWS_TAIL = N_NODES - NS * ROWS_PER_SUB

_sc_mesh = plsc.VectorSubcoreMesh(
    core_axis_name="c", subcore_axis_name="s", num_cores=NC, num_subcores=NS
)

_sc_params = pltpu.CompilerParams(use_tc_tiling_on_sc=False)


@functools.partial(
    pl.kernel,
    out_type=jax.ShapeDtypeStruct((NC, N_NODES, HIDDEN), jnp.float32),
    mesh=_sc_mesh,
    compiler_params=_sc_params,
    scratch_types=[
        pltpu.VMEM((NB + 1, CHUNK), jnp.int32),
        pltpu.VMEM((CHUNK, HIDDEN), jnp.float32),
        pltpu.VMEM((ROWS_PER_SUB, HIDDEN), jnp.float32),
        pltpu.VMEM_SHARED((N_NODES, HIDDEN), jnp.float32),
        pltpu.SemaphoreType.DMA,
    ],
)
def _hist(dst_hbm, out_hbm, idx_v, ones_v, zrows_v, acc_sh, sem):
    c = lax.axis_index("c")
    s = lax.axis_index("s")
    wid = s * NC + c

    @pl.loop(0, CHUNK)
    def _(i):
        ones_v[i, :] = jnp.ones((LANES,), jnp.float32)

    @pl.loop(0, ROWS_PER_SUB)
    def _(i):
        zrows_v[i, :] = jnp.zeros((LANES,), jnp.float32)

    pltpu.sync_copy(dst_hbm.at[pl.ds(wid * NB, NB)], idx_v.at[pl.ds(0, NB)])

    @pl.when(wid < EXTRA_CHUNKS)
    def _():
        pltpu.sync_copy(dst_hbm.at[pl.ds(NC * NS * NB + wid, 1)],
                        idx_v.at[pl.ds(NB, 1)])

    pltpu.sync_copy(zrows_v, acc_sh.at[pl.ds(s * ROWS_PER_SUB, ROWS_PER_SUB)])

    @pl.when(s == NS - 1)
    def _():
        pltpu.sync_copy(zrows_v.at[pl.ds(0, ROWS_TAIL)],
                        acc_sh.at[pl.ds(NS * ROWS_PER_SUB, ROWS_TAIL)])

    plsc.subcore_barrier()

    @pl.loop(0, NB)
    def _(j):
        pltpu.async_copy(ones_v, acc_sh.at[idx_v.at[j]], sem, add=True)

    @pl.when(wid < EXTRA_CHUNKS)
    def _():
        pltpu.async_copy(ones_v, acc_sh.at[idx_v.at[NB]], sem, add=True)

    @pl.loop(0, NB)
    def _(j):
        pltpu.make_async_copy(ones_v, acc_sh.at[idx_v.at[j]], sem).wait()

    @pl.when(wid < EXTRA_CHUNKS)
    def _():
        pltpu.make_async_copy(ones_v, acc_sh.at[idx_v.at[NB]], sem).wait()

    plsc.subcore_barrier()
    pltpu.sync_copy(
        acc_sh.at[pl.ds(s * ROWS_PER_SUB, ROWS_PER_SUB)],
        out_hbm.at[c, pl.ds(s * ROWS_PER_SUB, ROWS_PER_SUB)],
    )

    @pl.when(s == NS - 1)
    def _():
        pltpu.sync_copy(
            acc_sh.at[pl.ds(NS * ROWS_PER_SUB, ROWS_TAIL)],
            out_hbm.at[c, pl.ds(NS * ROWS_PER_SUB, ROWS_TAIL)],
        )


@functools.partial(
    pl.kernel,
    out_type=jax.ShapeDtypeStruct((NC, N_NODES, HIDDEN), jnp.float32),
    mesh=_sc_mesh,
    compiler_params=_sc_params,
    scratch_types=[
        pltpu.VMEM((NB + 1, CHUNK), jnp.int32),
        pltpu.VMEM((NB + 1, CHUNK), jnp.int32),
        pltpu.VMEM((CHUNK, HIDDEN), jnp.float32),
        pltpu.VMEM((CHUNK, HIDDEN), jnp.float32),
        pltpu.VMEM((ROWS_PER_SUB, HIDDEN), jnp.float32),
        pltpu.VMEM_SHARED((N_NODES, HIDDEN), jnp.float32),
        pltpu.SemaphoreType.DMA,
        pltpu.SemaphoreType.DMA,
    ],
)
def _segsum(src_hbm, dst_hbm, y_hbm, out_hbm,
            idx_s_v, idx_d_v, rows0_v, rows1_v, zrows_v, acc_sh, sem0, sem1):
    c = lax.axis_index("c")
    s = lax.axis_index("s")
    wid = s * NC + c

    @pl.loop(0, ROWS_PER_SUB)
    def _(i):
        zrows_v[i, :] = jnp.zeros((LANES,), jnp.float32)

    pltpu.sync_copy(src_hbm.at[pl.ds(wid * NB, NB)], idx_s_v.at[pl.ds(0, NB)])
    pltpu.sync_copy(dst_hbm.at[pl.ds(wid * NB, NB)], idx_d_v.at[pl.ds(0, NB)])

    @pl.when(wid < EXTRA_CHUNKS)
    def _():
        pltpu.sync_copy(src_hbm.at[pl.ds(NC * NS * NB + wid, 1)],
                        idx_s_v.at[pl.ds(NB, 1)])
        pltpu.sync_copy(dst_hbm.at[pl.ds(NC * NS * NB + wid, 1)],
                        idx_d_v.at[pl.ds(NB, 1)])

    rows_slc = pl.ds(s * ROWS_PER_SUB, ROWS_PER_SUB)
    tail_slc = pl.ds(NS * ROWS_PER_SUB, ROWS_TAIL)
    pltpu.sync_copy(zrows_v, acc_sh.at[rows_slc])

    @pl.when(s == NS - 1)
    def _():
        pltpu.sync_copy(zrows_v.at[pl.ds(0, ROWS_TAIL)], acc_sh.at[tail_slc])

    plsc.subcore_barrier()

    def gath(j, buf, sem):
        pltpu.async_copy(y_hbm.at[idx_s_v.at[j]], buf, sem)

    def wait_gath(j, buf, sem):
        pltpu.make_async_copy(y_hbm.at[idx_s_v.at[j]], buf, sem).wait()

    def scat(j, buf):
        pltpu.sync_copy(buf, acc_sh.at[idx_d_v.at[j]], add=True)

    gath(0, rows0_v, sem0)
    gath(1, rows1_v, sem1)

    @pl.loop(0, NPAIR)
    def _(p):
        j = 2 * p
        wait_gath(j, rows0_v, sem0)
        scat(j, rows0_v)

        @pl.when(p < NPAIR - 1)
        def _():
            gath(j + 2, rows0_v, sem0)

        wait_gath(j + 1, rows1_v, sem1)
        scat(j + 1, rows1_v)

        @pl.when(p < NPAIR - 1)
        def _():
            gath(j + 3, rows1_v, sem1)

    @pl.when(wid < EXTRA_CHUNKS)
    def _():
        gath(NB, rows0_v, sem0)
        wait_gath(NB, rows0_v, sem0)
        scat(NB, rows0_v)

    plsc.subcore_barrier()
    pltpu.sync_copy(acc_sh.at[rows_slc], out_hbm.at[c, rows_slc])

    @pl.when(s == NS - 1)
    def _():
        pltpu.sync_copy(acc_sh.at[tail_slc], out_hbm.at[c, tail_slc])


_BLK = 1000


def _dense_body(x_ref, w_ref, degp_ref, y_ref):
    deg = degp_ref[0] + degp_ref[1] + 1.0
    dis = lax.rsqrt(deg)
    xw = jnp.dot(x_ref[...], w_ref[...], preferred_element_type=jnp.float32)
    y_ref[...] = xw * dis


def _dense(x, w, degp):
    return pl.pallas_call(
        _dense_body,
        grid=(N_NODES // _BLK,),
        in_specs=[
            pl.BlockSpec((_BLK, D_FEAT), lambda i: (i, 0)),
            pl.BlockSpec((D_FEAT, HIDDEN), lambda i: (0, 0)),
            pl.BlockSpec((NC, _BLK, HIDDEN), lambda i: (0, i, 0)),
        ],
        out_specs=pl.BlockSpec((_BLK, HIDDEN), lambda i: (i, 0)),
        out_shape=jax.ShapeDtypeStruct((N_NODES, HIDDEN), jnp.float32),
    )(x, w, degp)


def _final_body(sp_ref, y_ref, degp_ref, b_ref, o_ref):
    deg = degp_ref[0] + degp_ref[1] + 1.0
    dis = lax.rsqrt(deg)
    h = dis * (sp_ref[0] + sp_ref[1] + y_ref[...]) + b_ref[...]
    h = jnp.maximum(h, 0.0)
    m = jnp.max(h, axis=1, keepdims=True)
    lse = jnp.log(jnp.sum(jnp.exp(h - m), axis=1, keepdims=True))
    o_ref[...] = h - m - lse


def _final(sp, y, degp, b):
    return pl.pallas_call(
        _final_body,
        grid=(N_NODES // _BLK,),
        in_specs=[
            pl.BlockSpec((NC, _BLK, HIDDEN), lambda i: (0, i, 0)),
            pl.BlockSpec((_BLK, HIDDEN), lambda i: (i, 0)),
            pl.BlockSpec((NC, _BLK, HIDDEN), lambda i: (0, i, 0)),
            pl.BlockSpec((1, HIDDEN), lambda i: (0, 0)),
        ],
        out_specs=pl.BlockSpec((_BLK, HIDDEN), lambda i: (i, 0)),
        out_shape=jax.ShapeDtypeStruct((N_NODES, HIDDEN), jnp.float32),
    )(sp, y, degp, b)


@jax.jit
def kernel(x, edge_index, W1, b1):
    src = edge_index[0].astype(jnp.int32).reshape(N_CHUNKS, CHUNK)
    dst = edge_index[1].astype(jnp.int32).reshape(N_CHUNKS, CHUNK)
    degp = _hist(dst)
    y = _dense(x, W1, degp)
    sp = _segsum(src, dst, y)
    return _final(sp, y, degp, b1.reshape(1, HIDDEN))


# speedup vs baseline: 61.7069x; 1.1508x over previous
import functools

import jax
import jax.numpy as jnp
from jax import lax
from jax.experimental import pallas as pl
from jax.experimental.pallas import tpu as pltpu
from jax.experimental.pallas import tpu_sc as plsc

N_NODES = 10000
N_EDGES = 320000
D_FEAT = 128
HIDDEN = 16

NC = 2
NS = 16
LANES = 16

CHUNK = 128
N_CHUNKS = N_EDGES // CHUNK
NB = N_CHUNKS // (NC * NS)
NBUF = 6
NGRP = NB // NBUF
EXTRA_CHUNKS = N_CHUNKS - NB * NC * NS
ROWS_PER_SUB = 624
ROWS_TAIL = N_NODES - NS * ROWS_PER_SUB

_sc_mesh = plsc.VectorSubcoreMesh(
    core_axis_name="c", subcore_axis_name="s", num_cores=NC, num_subcores=NS
)

_sc_params = pltpu.CompilerParams(use_tc_tiling_on_sc=False)


@functools.partial(
    pl.kernel,
    out_type=jax.ShapeDtypeStruct((NC, N_NODES, HIDDEN), jnp.float32),
    mesh=_sc_mesh,
    compiler_params=_sc_params,
    scratch_types=[
        pltpu.VMEM((NB + 1, CHUNK), jnp.int32),
        pltpu.VMEM((CHUNK, HIDDEN), jnp.float32),
        pltpu.VMEM((ROWS_PER_SUB, HIDDEN), jnp.float32),
        pltpu.VMEM_SHARED((N_NODES, HIDDEN), jnp.float32),
        pltpu.SemaphoreType.DMA,
    ],
)
def _hist(dst_hbm, out_hbm, idx_v, ones_v, zrows_v, acc_sh, sem):
    c = lax.axis_index("c")
    s = lax.axis_index("s")
    wid = s * NC + c

    @pl.loop(0, CHUNK)
    def _(i):
        ones_v[i, :] = jnp.ones((LANES,), jnp.float32)

    @pl.loop(0, ROWS_PER_SUB)
    def _(i):
        zrows_v[i, :] = jnp.zeros((LANES,), jnp.float32)

    pltpu.sync_copy(dst_hbm.at[pl.ds(wid * NB, NB)], idx_v.at[pl.ds(0, NB)])

    @pl.when(wid < EXTRA_CHUNKS)
    def _():
        pltpu.sync_copy(dst_hbm.at[pl.ds(NC * NS * NB + wid, 1)],
                        idx_v.at[pl.ds(NB, 1)])

    pltpu.sync_copy(zrows_v, acc_sh.at[pl.ds(s * ROWS_PER_SUB, ROWS_PER_SUB)])

    @pl.when(s == NS - 1)
    def _():
        pltpu.sync_copy(zrows_v.at[pl.ds(0, ROWS_TAIL)],
                        acc_sh.at[pl.ds(NS * ROWS_PER_SUB, ROWS_TAIL)])

    plsc.subcore_barrier()

    @pl.loop(0, NB)
    def _(j):
        pltpu.async_copy(ones_v, acc_sh.at[idx_v.at[j]], sem, add=True)

    @pl.when(wid < EXTRA_CHUNKS)
    def _():
        pltpu.async_copy(ones_v, acc_sh.at[idx_v.at[NB]], sem, add=True)

    @pl.loop(0, NB)
    def _(j):
        pltpu.make_async_copy(ones_v, acc_sh.at[idx_v.at[j]], sem).wait()

    @pl.when(wid < EXTRA_CHUNKS)
    def _():
        pltpu.make_async_copy(ones_v, acc_sh.at[idx_v.at[NB]], sem).wait()

    plsc.subcore_barrier()
    pltpu.sync_copy(
        acc_sh.at[pl.ds(s * ROWS_PER_SUB, ROWS_PER_SUB)],
        out_hbm.at[c, pl.ds(s * ROWS_PER_SUB, ROWS_PER_SUB)],
    )

    @pl.when(s == NS - 1)
    def _():
        pltpu.sync_copy(
            acc_sh.at[pl.ds(NS * ROWS_PER_SUB, ROWS_TAIL)],
            out_hbm.at[c, pl.ds(NS * ROWS_PER_SUB, ROWS_TAIL)],
        )


@functools.partial(
    pl.kernel,
    out_type=jax.ShapeDtypeStruct((NC, N_NODES, HIDDEN), jnp.float32),
    mesh=_sc_mesh,
    compiler_params=_sc_params,
    scratch_types=[
        pltpu.VMEM((NB + 1, CHUNK), jnp.int32),
        pltpu.VMEM((NB + 1, CHUNK), jnp.int32),
        [pltpu.VMEM((CHUNK, HIDDEN), jnp.float32) for _ in range(NBUF)],
        pltpu.VMEM((ROWS_PER_SUB, HIDDEN), jnp.float32),
        pltpu.VMEM_SHARED((N_NODES, HIDDEN), jnp.float32),
        [pltpu.SemaphoreType.DMA for _ in range(NBUF)],
    ],
)
def _segsum(src_hbm, dst_hbm, y_hbm, out_hbm,
            idx_s_v, idx_d_v, rows_bufs, zrows_v, acc_sh, sems):
    c = lax.axis_index("c")
    s = lax.axis_index("s")
    wid = s * NC + c

    @pl.loop(0, ROWS_PER_SUB)
    def _(i):
        zrows_v[i, :] = jnp.zeros((LANES,), jnp.float32)

    pltpu.sync_copy(src_hbm.at[pl.ds(wid * NB, NB)], idx_s_v.at[pl.ds(0, NB)])
    pltpu.sync_copy(dst_hbm.at[pl.ds(wid * NB, NB)], idx_d_v.at[pl.ds(0, NB)])

    @pl.when(wid < EXTRA_CHUNKS)
    def _():
        pltpu.sync_copy(src_hbm.at[pl.ds(NC * NS * NB + wid, 1)],
                        idx_s_v.at[pl.ds(NB, 1)])
        pltpu.sync_copy(dst_hbm.at[pl.ds(NC * NS * NB + wid, 1)],
                        idx_d_v.at[pl.ds(NB, 1)])

    rows_slc = pl.ds(s * ROWS_PER_SUB, ROWS_PER_SUB)
    tail_slc = pl.ds(NS * ROWS_PER_SUB, ROWS_TAIL)
    pltpu.sync_copy(zrows_v, acc_sh.at[rows_slc])

    @pl.when(s == NS - 1)
    def _():
        pltpu.sync_copy(zrows_v.at[pl.ds(0, ROWS_TAIL)], acc_sh.at[tail_slc])

    plsc.subcore_barrier()

    def gath(j, buf, sem):
        pltpu.async_copy(y_hbm.at[idx_s_v.at[j]], buf, sem)

    def wait_one(buf, sem):
        pltpu.make_async_copy(y_hbm.at[idx_s_v.at[0]], buf, sem).wait()

    def scat(j, buf, sem):
        pltpu.async_copy(buf, acc_sh.at[idx_d_v.at[j]], sem, add=True)

    for b in range(NBUF):
        gath(b, rows_bufs[b], sems[b])

    @pl.loop(0, NGRP)
    def _(g):
        j0 = g * NBUF
        for b in range(NBUF):
            wait_one(rows_bufs[b], sems[b])
            scat(j0 + b, rows_bufs[b], sems[b])

        @pl.when(g < NGRP - 1)
        def _():
            for b in range(NBUF):
                wait_one(rows_bufs[b], sems[b])
                gath(j0 + NBUF + b, rows_bufs[b], sems[b])

    for b in range(NBUF):
        wait_one(rows_bufs[b], sems[b])

    @pl.when(wid < EXTRA_CHUNKS)
    def _():
        gath(NB, rows_bufs[0], sems[0])
        wait_one(rows_bufs[0], sems[0])
        scat(NB, rows_bufs[0], sems[0])
        wait_one(rows_bufs[0], sems[0])

    plsc.subcore_barrier()
    pltpu.sync_copy(acc_sh.at[rows_slc], out_hbm.at[c, rows_slc])

    @pl.when(s == NS - 1)
    def _():
        pltpu.sync_copy(acc_sh.at[tail_slc], out_hbm.at[c, tail_slc])


_BLK = 1000


def _dense_body(x_ref, w_ref, degp_ref, y_ref):
    deg = degp_ref[0] + degp_ref[1] + 1.0
    dis = lax.rsqrt(deg)
    xw = jnp.dot(x_ref[...], w_ref[...], preferred_element_type=jnp.float32)
    y_ref[...] = xw * dis


def _dense(x, w, degp):
    return pl.pallas_call(
        _dense_body,
        grid=(N_NODES // _BLK,),
        in_specs=[
            pl.BlockSpec((_BLK, D_FEAT), lambda i: (i, 0)),
            pl.BlockSpec((D_FEAT, HIDDEN), lambda i: (0, 0)),
            pl.BlockSpec((NC, _BLK, HIDDEN), lambda i: (0, i, 0)),
        ],
        out_specs=pl.BlockSpec((_BLK, HIDDEN), lambda i: (i, 0)),
        out_shape=jax.ShapeDtypeStruct((N_NODES, HIDDEN), jnp.float32),
    )(x, w, degp)


def _final_body(sp_ref, y_ref, degp_ref, b_ref, o_ref):
    deg = degp_ref[0] + degp_ref[1] + 1.0
    dis = lax.rsqrt(deg)
    h = dis * (sp_ref[0] + sp_ref[1] + y_ref[...]) + b_ref[...]
    h = jnp.maximum(h, 0.0)
    m = jnp.max(h, axis=1, keepdims=True)
    lse = jnp.log(jnp.sum(jnp.exp(h - m), axis=1, keepdims=True))
    o_ref[...] = h - m - lse


def _final(sp, y, degp, b):
    return pl.pallas_call(
        _final_body,
        grid=(N_NODES // _BLK,),
        in_specs=[
            pl.BlockSpec((NC, _BLK, HIDDEN), lambda i: (0, i, 0)),
            pl.BlockSpec((_BLK, HIDDEN), lambda i: (i, 0)),
            pl.BlockSpec((NC, _BLK, HIDDEN), lambda i: (0, i, 0)),
            pl.BlockSpec((1, HIDDEN), lambda i: (0, 0)),
        ],
        out_specs=pl.BlockSpec((_BLK, HIDDEN), lambda i: (i, 0)),
        out_shape=jax.ShapeDtypeStruct((N_NODES, HIDDEN), jnp.float32),
    )(sp, y, degp, b)


@jax.jit
def kernel(x, edge_index, W1, b1):
    src = edge_index[0].astype(jnp.int32).reshape(N_CHUNKS, CHUNK)
    dst = edge_index[1].astype(jnp.int32).reshape(N_CHUNKS, CHUNK)
    degp = _hist(dst)
    y = _dense(x, W1, degp)
    sp = _segsum(src, dst, y)
    return _final(sp, y, degp, b1.reshape(1, HIDDEN))
